# baseline (speedup 1.0000x reference)
import functools

import jax
import jax.numpy as jnp
from jax import lax
from jax.experimental import pallas as pl
from jax.experimental.pallas import tpu as pltpu
from jax.experimental.pallas import tpu_sc as plsc

N = 10000
E = 320000
D = 128
H = D // 2
G = 64

NC = 2
NS = 16
BATCH = 128
NBUF = 4
LOOK = 2
EPW = (E + NS - 1) // NS
NB = -(-EPW // (BATCH * NBUF)) * NBUF
EPAD = NS * NB * BATCH
CH = 640
NPAD = NS * CH
DUMMY = N

BLK = 2000


def _sc_agg_body(x_hbm, src_hbm, dst_hbm, z2_hbm, z1_hbm, agg_out, cnt_out,
                 src_v, dst_v, r0, r1, r2, r3, ones_v, agg_sh, cnt_sh,
                 g0, g1, g2, g3):
    rows = (r0, r1, r2, r3)
    gsem = (g0, g1, g2, g3)
    c = lax.axis_index("c")
    s = lax.axis_index("s")

    pltpu.sync_copy(z2_hbm, agg_sh.at[pl.ds(s * CH, CH)])
    pltpu.sync_copy(src_hbm.at[c, s], src_v)
    pltpu.sync_copy(dst_hbm.at[s], dst_v)

    @pl.when(c == 0)
    def _():
        pltpu.sync_copy(z1_hbm, cnt_sh.at[pl.ds(s * CH, CH)])

    for i in range(BATCH // 16):
        ones_v[pl.ds(i * 16, 16)] = jnp.full((16,), 1.0, jnp.float32)
    for b in range(LOOK):
        pltpu.async_copy(x_hbm.at[src_v.at[b]], rows[b], gsem[b])
    plsc.subcore_barrier()

    @pl.loop(0, NB, step=NBUF)
    def _(gbase):
        for b in range(NBUF):
            i = gbase + b
            bn = (b + LOOK) % NBUF
            pltpu.make_async_copy(x_hbm.at[src_v.at[i]], rows[b],
                                  gsem[b]).wait()

            @pl.when(i + LOOK < NB)
            def _():
                pltpu.async_copy(x_hbm.at[src_v.at[i + LOOK]], rows[bn],
                                 gsem[bn])

            pltpu.sync_copy(rows[b], agg_sh.at[dst_v.at[i]], add=True)

            @pl.when(c == 0)
            def _():
                pltpu.sync_copy(ones_v, cnt_sh.at[dst_v.at[i]], add=True)

    plsc.subcore_barrier()
    pltpu.sync_copy(agg_sh.at[pl.ds(s * CH, CH)],
                    agg_out.at[pl.ds(s * CH, CH), pl.ds(c * H, H)])

    @pl.when(c == 0)
    def _():
        pltpu.sync_copy(cnt_sh.at[pl.ds(s * CH, CH)],
                        cnt_out.at[pl.ds(s * CH, CH)])


_sc_agg = functools.partial(
    pl.kernel,
    out_type=[
        jax.ShapeDtypeStruct((NPAD, D), jnp.float32),
        jax.ShapeDtypeStruct((NPAD,), jnp.float32),
    ],
    mesh=plsc.VectorSubcoreMesh(core_axis_name="c", subcore_axis_name="s"),
    compiler_params=pltpu.CompilerParams(use_tc_tiling_on_sc=False,
                                         disable_bounds_checks=True),
    scratch_types=(
        [
            pltpu.VMEM((NB, BATCH), jnp.int32),
            pltpu.VMEM((NB, BATCH), jnp.int32),
        ]
        + [pltpu.VMEM((BATCH, H), jnp.float32)] * NBUF
        + [
            pltpu.VMEM((BATCH,), jnp.float32),
            pltpu.VMEM_SHARED((NPAD, H), jnp.float32),
            pltpu.VMEM_SHARED((NPAD,), jnp.float32),
        ]
        + [pltpu.SemaphoreType.DMA] * NBUF
    ),
)(_sc_agg_body)


def _layer1_body(agg_ref, cnt_ref, y_ref, wl_ref, wr_ref, b_ref, out_ref):
    rcp = 1.0 / jnp.maximum(cnt_ref[:], 1.0)
    out = (lax.dot_general(agg_ref[:] * rcp, wl_ref[:],
                           (((1,), (1,)), ((), ())),
                           preferred_element_type=jnp.float32)
           + lax.dot_general(y_ref[:], wr_ref[:], (((1,), (1,)), ((), ())),
                             preferred_element_type=jnp.float32)
           + b_ref[:])
    out_ref[:] = jnp.where(out >= 0.0, out, 0.01 * out)


def _layer2_body(agg_ref, cnt_ref, y_ref, wl_ref, wr_ref, b_ref, bat_ref,
                 node_ref, graph_ref):
    i = pl.program_id(0)
    rcp = 1.0 / jnp.maximum(cnt_ref[:], 1.0)
    nm = (lax.dot_general(agg_ref[:] * rcp, wl_ref[:],
                          (((1,), (1,)), ((), ())),
                          preferred_element_type=jnp.float32)
          + lax.dot_general(y_ref[:], wr_ref[:], (((1,), (1,)), ((), ())),
                            preferred_element_type=jnp.float32)
          + b_ref[:])
    node_ref[:] = nm
    onehot = (bat_ref[:] == lax.broadcasted_iota(jnp.int32, (BLK, G), 1)
              ).astype(jnp.float32)
    contrib = lax.dot_general(onehot, nm, (((0,), (0,)), ((), ())),
                              preferred_element_type=jnp.float32)

    @pl.when(i == 0)
    def _():
        graph_ref[:] = contrib

    @pl.when(i > 0)
    def _():
        graph_ref[:] += contrib


_COMMON_SPECS = [
    pl.BlockSpec((BLK, D), lambda i: (i, 0)),
    pl.BlockSpec((BLK, 1), lambda i: (i, 0)),
    pl.BlockSpec((BLK, D), lambda i: (i, 0)),
    pl.BlockSpec((D, D), lambda i: (0, 0)),
    pl.BlockSpec((D, D), lambda i: (0, 0)),
    pl.BlockSpec((1, D), lambda i: (0, 0)),
]

_layer1 = pl.pallas_call(
    _layer1_body,
    grid=(N // BLK,),
    in_specs=_COMMON_SPECS,
    out_specs=pl.BlockSpec((BLK, D), lambda i: (i, 0)),
    out_shape=jax.ShapeDtypeStruct((N, D), jnp.float32),
)

_layer2 = pl.pallas_call(
    _layer2_body,
    grid=(N // BLK,),
    in_specs=_COMMON_SPECS + [pl.BlockSpec((BLK, 1), lambda i: (i, 0))],
    out_specs=[
        pl.BlockSpec((BLK, D), lambda i: (i, 0)),
        pl.BlockSpec((G, D), lambda i: (0, 0)),
    ],
    out_shape=[
        jax.ShapeDtypeStruct((N, D), jnp.float32),
        jax.ShapeDtypeStruct((G, D), jnp.float32),
    ],
)


def kernel(x, edge_index, batch, W1_l, W1_r, b1, W2_l, W2_r, b2):
    ei = edge_index.reshape(2 * E)
    src = ei[:E]
    dst = ei[E:]
    pad = EPAD - E
    src_pad = jnp.arange(pad, dtype=jnp.int32) % N
    dst_pad = DUMMY + jnp.arange(pad, dtype=jnp.int32) % (NPAD - N)
    src_p = jnp.concatenate([src, src_pad]).reshape(NS, NB, BATCH)
    src4 = jnp.stack([2 * src_p, 2 * src_p + 1])
    dst3 = jnp.concatenate([dst, dst_pad]).reshape(NS, NB, BATCH)
    z2 = jnp.zeros((CH, H), jnp.float32)
    z1 = jnp.zeros((CH,), jnp.float32)

    agg1, cnt = _sc_agg(x.reshape(2 * N, H), src4, dst3, z2, z1)
    cnt2 = cnt[:N].reshape(N, 1)
    h = _layer1(agg1, cnt2, x, W1_l, W1_r, b1.reshape(1, D))
    agg2, _ = _sc_agg(h.reshape(2 * N, H), src4, dst3, z2, z1)
    node_emb, graph_emb = _layer2(agg2, cnt2, h, W2_l, W2_r,
                                  b2.reshape(1, D), batch.reshape(N, 1))
    return node_emb, graph_emb


# speedup vs baseline: 1.0160x; 1.0160x over previous
import functools

import jax
import jax.numpy as jnp
from jax import lax
from jax.experimental import pallas as pl
from jax.experimental.pallas import tpu as pltpu
from jax.experimental.pallas import tpu_sc as plsc

N = 10000
E = 320000
D = 128
H = D // 2
G = 64

NC = 2
NS = 16
BATCH = 128
NBUF = 4
LOOK = 2
EPW = (E + NS - 1) // NS
NB = -(-EPW // (BATCH * NBUF)) * NBUF
EPAD = NS * NB * BATCH
CH = 640
NPAD = NS * CH
DUMMY = N

BLK = 2000


def _sc_agg_body(x_hbm, src_hbm, dst_hbm, z2_hbm, z1_hbm, agg_out, cnt_out,
                 src_v, dst_v, r0, r1, r2, r3, ones_v, agg_sh, cnt_sh,
                 g0, g1, g2, g3):
    rows = (r0, r1, r2, r3)
    gsem = (g0, g1, g2, g3)
    c = lax.axis_index("c")
    s = lax.axis_index("s")

    pltpu.sync_copy(z2_hbm, agg_sh.at[pl.ds(s * CH, CH)])
    pltpu.sync_copy(src_hbm.at[c, s], src_v)
    pltpu.sync_copy(dst_hbm.at[s], dst_v)

    @pl.when(c == 0)
    def _():
        pltpu.sync_copy(z1_hbm, cnt_sh.at[pl.ds(s * CH, CH)])

    for i in range(BATCH // 16):
        ones_v[pl.ds(i * 16, 16)] = jnp.full((16,), 1.0, jnp.float32)
    for b in range(LOOK):
        pltpu.async_copy(x_hbm.at[src_v.at[b]], rows[b], gsem[b])
    plsc.subcore_barrier()

    @pl.loop(0, NB, step=NBUF)
    def _(gbase):
        for b in range(NBUF):
            i = gbase + b
            bn = (b + LOOK) % NBUF
            pltpu.make_async_copy(x_hbm.at[src_v.at[i]], rows[b],
                                  gsem[b]).wait()

            @pl.when(i + LOOK < NB)
            def _():
                pltpu.async_copy(x_hbm.at[src_v.at[i + LOOK]], rows[bn],
                                 gsem[bn])

            pltpu.sync_copy(rows[b], agg_sh.at[dst_v.at[i]], add=True)

            @pl.when(c == 0)
            def _():
                pltpu.sync_copy(ones_v, cnt_sh.at[dst_v.at[i]], add=True)

    plsc.subcore_barrier()
    pltpu.sync_copy(agg_sh.at[pl.ds(s * CH, CH)],
                    agg_out.at[pl.ds(s * CH, CH), pl.ds(c * H, H)])

    @pl.when(c == 0)
    def _():
        pltpu.sync_copy(cnt_sh.at[pl.ds(s * CH, CH)],
                        cnt_out.at[pl.ds(s * CH, CH)])


_sc_agg = functools.partial(
    pl.kernel,
    out_type=[
        jax.ShapeDtypeStruct((NPAD, D), jnp.float32),
        jax.ShapeDtypeStruct((NPAD,), jnp.float32),
    ],
    mesh=plsc.VectorSubcoreMesh(core_axis_name="c", subcore_axis_name="s"),
    compiler_params=pltpu.CompilerParams(use_tc_tiling_on_sc=False,
                                         disable_bounds_checks=True),
    scratch_types=(
        [
            pltpu.VMEM((NB, BATCH), jnp.int32),
            pltpu.VMEM((NB, BATCH), jnp.int32),
        ]
        + [pltpu.VMEM((BATCH, H), jnp.float32)] * NBUF
        + [
            pltpu.VMEM((BATCH,), jnp.float32),
            pltpu.VMEM_SHARED((NPAD, H), jnp.float32),
            pltpu.VMEM_SHARED((NPAD,), jnp.float32),
        ]
        + [pltpu.SemaphoreType.DMA] * NBUF
    ),
)(_sc_agg_body)


def _layer1_body(agg_ref, cnt_ref, y_ref, wl_ref, wr_ref, b_ref, out_ref):
    rcp = 1.0 / jnp.maximum(cnt_ref[:], 1.0)
    out = (lax.dot_general(agg_ref[:] * rcp, wl_ref[:],
                           (((1,), (1,)), ((), ())),
                           preferred_element_type=jnp.float32)
           + lax.dot_general(y_ref[:], wr_ref[:], (((1,), (1,)), ((), ())),
                             preferred_element_type=jnp.float32)
           + b_ref[:])
    out_ref[:] = jnp.where(out >= 0.0, out, 0.01 * out)


def _layer2_body(agg_ref, cnt_ref, y_ref, wl_ref, wr_ref, b_ref, bat_ref,
                 node_ref, graph_ref):
    i = pl.program_id(0)
    rcp = 1.0 / jnp.maximum(cnt_ref[:], 1.0)
    nm = (lax.dot_general(agg_ref[:] * rcp, wl_ref[:],
                          (((1,), (1,)), ((), ())),
                          preferred_element_type=jnp.float32)
          + lax.dot_general(y_ref[:], wr_ref[:], (((1,), (1,)), ((), ())),
                            preferred_element_type=jnp.float32)
          + b_ref[:])
    node_ref[:] = nm
    onehot = (bat_ref[:] == lax.broadcasted_iota(jnp.int32, (BLK, G), 1)
              ).astype(jnp.float32)
    contrib = lax.dot_general(onehot, nm, (((0,), (0,)), ((), ())),
                              preferred_element_type=jnp.float32)

    @pl.when(i == 0)
    def _():
        graph_ref[:] = contrib

    @pl.when(i > 0)
    def _():
        graph_ref[:] += contrib


_COMMON_SPECS = [
    pl.BlockSpec((BLK, D), lambda i: (i, 0)),
    pl.BlockSpec((BLK, 1), lambda i: (i, 0)),
    pl.BlockSpec((BLK, D), lambda i: (i, 0)),
    pl.BlockSpec((D, D), lambda i: (0, 0)),
    pl.BlockSpec((D, D), lambda i: (0, 0)),
    pl.BlockSpec((1, D), lambda i: (0, 0)),
]

_layer1 = pl.pallas_call(
    _layer1_body,
    grid=(N // BLK,),
    in_specs=_COMMON_SPECS,
    out_specs=pl.BlockSpec((BLK, D), lambda i: (i, 0)),
    out_shape=jax.ShapeDtypeStruct((N, D), jnp.float32),
)

_layer2 = pl.pallas_call(
    _layer2_body,
    grid=(N // BLK,),
    in_specs=_COMMON_SPECS + [pl.BlockSpec((BLK, 1), lambda i: (i, 0))],
    out_specs=[
        pl.BlockSpec((BLK, D), lambda i: (i, 0)),
        pl.BlockSpec((G, D), lambda i: (0, 0)),
    ],
    out_shape=[
        jax.ShapeDtypeStruct((N, D), jnp.float32),
        jax.ShapeDtypeStruct((G, D), jnp.float32),
    ],
)


def kernel(x, edge_index, batch, W1_l, W1_r, b1, W2_l, W2_r, b2):
    src = edge_index[0]
    dst = edge_index[1]
    pad = EPAD - E
    src_pad = jnp.arange(pad, dtype=jnp.int32) % N
    dst_pad = DUMMY + jnp.arange(pad, dtype=jnp.int32) % (NPAD - N)
    src_p = jnp.concatenate([src, src_pad]).reshape(NS, NB, BATCH)
    src4 = jnp.stack([2 * src_p, 2 * src_p + 1])
    dst3 = jnp.concatenate([dst, dst_pad]).reshape(NS, NB, BATCH)
    z2 = jnp.zeros((CH, H), jnp.float32)
    z1 = jnp.zeros((CH,), jnp.float32)

    agg1, cnt = _sc_agg(x.reshape(2 * N, H), src4, dst3, z2, z1)
    cnt2 = cnt[:N].reshape(N, 1)
    h = _layer1(agg1, cnt2, x, W1_l, W1_r, b1.reshape(1, D))
    agg2, _ = _sc_agg(h.reshape(2 * N, H), src4, dst3, z2, z1)
    node_emb, graph_emb = _layer2(agg2, cnt2, h, W2_l, W2_r,
                                  b2.reshape(1, D), batch.reshape(N, 1))
    return node_emb, graph_emb


# speedup vs baseline: 1.0585x; 1.0418x over previous
import functools

import jax
import jax.numpy as jnp
from jax import lax
from jax.experimental import pallas as pl
from jax.experimental.pallas import tpu as pltpu
from jax.experimental.pallas import tpu_sc as plsc

N = 10000
E = 320000
D = 128
H = D // 2
G = 64

NC = 2
NS = 16
BATCH = 128
NBUF = 4
LOOK = 3
EPW = (E + NS - 1) // NS
NB = -(-EPW // (BATCH * NBUF)) * NBUF
EPAD = NS * NB * BATCH
CH = 640
NPAD = NS * CH
DUMMY = N

BLK = 2000


def _sc_agg_body(x_hbm, src_hbm, dst_hbm, z2_hbm, z1_hbm, agg_out, cnt_out,
                 src_v, dst_v, r0, r1, r2, r3, ones_v, agg_sh, cnt_sh,
                 g0, g1, g2, g3):
    rows = (r0, r1, r2, r3)
    gsem = (g0, g1, g2, g3)
    c = lax.axis_index("c")
    s = lax.axis_index("s")

    pltpu.sync_copy(z2_hbm, agg_sh.at[pl.ds(s * CH, CH)])
    pltpu.sync_copy(src_hbm.at[c, s], src_v)
    pltpu.sync_copy(dst_hbm.at[s], dst_v)

    @pl.when(c == 0)
    def _():
        pltpu.sync_copy(z1_hbm, cnt_sh.at[pl.ds(s * CH, CH)])

    for i in range(BATCH // 16):
        ones_v[pl.ds(i * 16, 16)] = jnp.full((16,), 1.0, jnp.float32)
    for b in range(LOOK):
        pltpu.async_copy(x_hbm.at[src_v.at[b]], rows[b], gsem[b])
    plsc.subcore_barrier()

    @pl.loop(0, NB, step=NBUF)
    def _(gbase):
        for b in range(NBUF):
            i = gbase + b
            bn = (b + LOOK) % NBUF
            pltpu.make_async_copy(x_hbm.at[src_v.at[i]], rows[b],
                                  gsem[b]).wait()

            @pl.when(i + LOOK < NB)
            def _():
                pltpu.async_copy(x_hbm.at[src_v.at[i + LOOK]], rows[bn],
                                 gsem[bn])

            pltpu.sync_copy(rows[b], agg_sh.at[dst_v.at[i]], add=True)

            @pl.when(c == 0)
            def _():
                pltpu.sync_copy(ones_v, cnt_sh.at[dst_v.at[i]], add=True)

    plsc.subcore_barrier()
    pltpu.sync_copy(agg_sh.at[pl.ds(s * CH, CH)],
                    agg_out.at[pl.ds(s * CH, CH), pl.ds(c * H, H)])

    @pl.when(c == 0)
    def _():
        pltpu.sync_copy(cnt_sh.at[pl.ds(s * CH, CH)],
                        cnt_out.at[pl.ds(s * CH, CH)])


_sc_agg = functools.partial(
    pl.kernel,
    out_type=[
        jax.ShapeDtypeStruct((NPAD, D), jnp.float32),
        jax.ShapeDtypeStruct((NPAD,), jnp.float32),
    ],
    mesh=plsc.VectorSubcoreMesh(core_axis_name="c", subcore_axis_name="s"),
    compiler_params=pltpu.CompilerParams(use_tc_tiling_on_sc=False,
                                         disable_bounds_checks=True),
    scratch_types=(
        [
            pltpu.VMEM((NB, BATCH), jnp.int32),
            pltpu.VMEM((NB, BATCH), jnp.int32),
        ]
        + [pltpu.VMEM((BATCH, H), jnp.float32)] * NBUF
        + [
            pltpu.VMEM((BATCH,), jnp.float32),
            pltpu.VMEM_SHARED((NPAD, H), jnp.float32),
            pltpu.VMEM_SHARED((NPAD,), jnp.float32),
        ]
        + [pltpu.SemaphoreType.DMA] * NBUF
    ),
)(_sc_agg_body)


def _layer1_body(agg_ref, cnt_ref, y_ref, wl_ref, wr_ref, b_ref, out_ref):
    rcp = 1.0 / jnp.maximum(cnt_ref[:], 1.0)
    out = (lax.dot_general(agg_ref[:] * rcp, wl_ref[:],
                           (((1,), (1,)), ((), ())),
                           preferred_element_type=jnp.float32)
           + lax.dot_general(y_ref[:], wr_ref[:], (((1,), (1,)), ((), ())),
                             preferred_element_type=jnp.float32)
           + b_ref[:])
    out_ref[:] = jnp.where(out >= 0.0, out, 0.01 * out)


def _layer2_body(agg_ref, cnt_ref, y_ref, wl_ref, wr_ref, b_ref, bat_ref,
                 node_ref, graph_ref):
    i = pl.program_id(0)
    rcp = 1.0 / jnp.maximum(cnt_ref[:], 1.0)
    nm = (lax.dot_general(agg_ref[:] * rcp, wl_ref[:],
                          (((1,), (1,)), ((), ())),
                          preferred_element_type=jnp.float32)
          + lax.dot_general(y_ref[:], wr_ref[:], (((1,), (1,)), ((), ())),
                            preferred_element_type=jnp.float32)
          + b_ref[:])
    node_ref[:] = nm
    onehot = (bat_ref[:] == lax.broadcasted_iota(jnp.int32, (BLK, G), 1)
              ).astype(jnp.float32)
    contrib = lax.dot_general(onehot, nm, (((0,), (0,)), ((), ())),
                              preferred_element_type=jnp.float32)

    @pl.when(i == 0)
    def _():
        graph_ref[:] = contrib

    @pl.when(i > 0)
    def _():
        graph_ref[:] += contrib


_COMMON_SPECS = [
    pl.BlockSpec((BLK, D), lambda i: (i, 0)),
    pl.BlockSpec((BLK, 1), lambda i: (i, 0)),
    pl.BlockSpec((BLK, D), lambda i: (i, 0)),
    pl.BlockSpec((D, D), lambda i: (0, 0)),
    pl.BlockSpec((D, D), lambda i: (0, 0)),
    pl.BlockSpec((1, D), lambda i: (0, 0)),
]

_layer1 = pl.pallas_call(
    _layer1_body,
    grid=(N // BLK,),
    in_specs=_COMMON_SPECS,
    out_specs=pl.BlockSpec((BLK, D), lambda i: (i, 0)),
    out_shape=jax.ShapeDtypeStruct((N, D), jnp.float32),
)

_layer2 = pl.pallas_call(
    _layer2_body,
    grid=(N // BLK,),
    in_specs=_COMMON_SPECS + [pl.BlockSpec((BLK, 1), lambda i: (i, 0))],
    out_specs=[
        pl.BlockSpec((BLK, D), lambda i: (i, 0)),
        pl.BlockSpec((G, D), lambda i: (0, 0)),
    ],
    out_shape=[
        jax.ShapeDtypeStruct((N, D), jnp.float32),
        jax.ShapeDtypeStruct((G, D), jnp.float32),
    ],
)


def kernel(x, edge_index, batch, W1_l, W1_r, b1, W2_l, W2_r, b2):
    src = edge_index[0]
    dst = edge_index[1]
    pad = EPAD - E
    src_pad = jnp.arange(pad, dtype=jnp.int32) % N
    dst_pad = DUMMY + jnp.arange(pad, dtype=jnp.int32) % (NPAD - N)
    src_p = jnp.concatenate([src, src_pad]).reshape(NS, NB, BATCH)
    src4 = jnp.stack([2 * src_p, 2 * src_p + 1])
    dst3 = jnp.concatenate([dst, dst_pad]).reshape(NS, NB, BATCH)
    z2 = jnp.zeros((CH, H), jnp.float32)
    z1 = jnp.zeros((CH,), jnp.float32)

    agg1, cnt = _sc_agg(x.reshape(2 * N, H), src4, dst3, z2, z1)
    cnt2 = cnt[:N].reshape(N, 1)
    h = _layer1(agg1, cnt2, x, W1_l, W1_r, b1.reshape(1, D))
    agg2, _ = _sc_agg(h.reshape(2 * N, H), src4, dst3, z2, z1)
    node_emb, graph_emb = _layer2(agg2, cnt2, h, W2_l, W2_r,
                                  b2.reshape(1, D), batch.reshape(N, 1))
    return node_emb, graph_emb


# speedup vs baseline: 1.0801x; 1.0204x over previous
import functools

import jax
import jax.numpy as jnp
from jax import lax
from jax.experimental import pallas as pl
from jax.experimental.pallas import tpu as pltpu
from jax.experimental.pallas import tpu_sc as plsc

N = 10000
E = 320000
D = 128
H = D // 2
G = 64

NC = 2
NS = 16
BATCH = 128
NBUF = 4
LOOK = 3
EPW = (E + NS - 1) // NS
NB = -(-EPW // (BATCH * NBUF)) * NBUF
EPAD = NS * NB * BATCH
CH = 640
NPAD = NS * CH
DUMMY = N

BLK = 2000


def _make_sc_agg(with_cnt):
    def body(*refs):
        if with_cnt:
            (x_hbm, src_hbm, dst_hbm, z2_hbm, z1_hbm, agg_out, cnt_out,
             src_v, dst_v, r0, r1, r2, r3, ones_v, agg_sh, cnt_sh,
             g0, g1, g2, g3) = refs
        else:
            (x_hbm, src_hbm, dst_hbm, z2_hbm, agg_out,
             src_v, dst_v, r0, r1, r2, r3, agg_sh,
             g0, g1, g2, g3) = refs
        rows = (r0, r1, r2, r3)
        gsem = (g0, g1, g2, g3)
        c = lax.axis_index("c")
        s = lax.axis_index("s")

        pltpu.sync_copy(z2_hbm, agg_sh.at[pl.ds(s * CH, CH)])
        pltpu.sync_copy(src_hbm.at[c, s], src_v)
        pltpu.sync_copy(dst_hbm.at[s], dst_v)

        if with_cnt:
            @pl.when(c == 0)
            def _():
                pltpu.sync_copy(z1_hbm, cnt_sh.at[pl.ds(s * CH, CH)])

            for i in range(BATCH // 16):
                ones_v[pl.ds(i * 16, 16)] = jnp.full((16,), 1.0, jnp.float32)

        for b in range(LOOK):
            pltpu.async_copy(x_hbm.at[src_v.at[b]], rows[b], gsem[b])
        plsc.subcore_barrier()

        @pl.loop(0, NB, step=NBUF)
        def _(gbase):
            for b in range(NBUF):
                i = gbase + b
                bn = (b + LOOK) % NBUF
                pltpu.make_async_copy(x_hbm.at[src_v.at[i]], rows[b],
                                      gsem[b]).wait()

                @pl.when(i + LOOK < NB)
                def _():
                    pltpu.async_copy(x_hbm.at[src_v.at[i + LOOK]], rows[bn],
                                     gsem[bn])

                pltpu.sync_copy(rows[b], agg_sh.at[dst_v.at[i]], add=True)

                if with_cnt:
                    @pl.when(c == 0)
                    def _():
                        pltpu.sync_copy(ones_v, cnt_sh.at[dst_v.at[i]],
                                        add=True)

        plsc.subcore_barrier()
        pltpu.sync_copy(agg_sh.at[pl.ds(s * CH, CH)],
                        agg_out.at[pl.ds(s * CH, CH), pl.ds(c * H, H)])

        if with_cnt:
            @pl.when(c == 0)
            def _():
                pltpu.sync_copy(cnt_sh.at[pl.ds(s * CH, CH)],
                                cnt_out.at[pl.ds(s * CH, CH)])

    out_type = jax.ShapeDtypeStruct((NPAD, D), jnp.float32)
    if with_cnt:
        out_type = [out_type, jax.ShapeDtypeStruct((NPAD,), jnp.float32)]
    scratch = [
        pltpu.VMEM((NB, BATCH), jnp.int32),
        pltpu.VMEM((NB, BATCH), jnp.int32),
    ] + [pltpu.VMEM((BATCH, H), jnp.float32)] * NBUF
    if with_cnt:
        scratch.append(pltpu.VMEM((BATCH,), jnp.float32))
    scratch.append(pltpu.VMEM_SHARED((NPAD, H), jnp.float32))
    if with_cnt:
        scratch.append(pltpu.VMEM_SHARED((NPAD,), jnp.float32))
    scratch += [pltpu.SemaphoreType.DMA] * NBUF
    return pl.kernel(
        body,
        out_type=out_type,
        mesh=plsc.VectorSubcoreMesh(core_axis_name="c", subcore_axis_name="s"),
        compiler_params=pltpu.CompilerParams(use_tc_tiling_on_sc=False,
                                             disable_bounds_checks=True),
        scratch_types=scratch,
    )


_sc_agg_cnt = _make_sc_agg(True)
_sc_agg_nocnt = _make_sc_agg(False)


def _layer1_body(agg_ref, cnt_ref, y_ref, wl_ref, wr_ref, b_ref, out_ref):
    rcp = 1.0 / jnp.maximum(cnt_ref[:], 1.0)
    out = (lax.dot_general(agg_ref[:] * rcp, wl_ref[:],
                           (((1,), (1,)), ((), ())),
                           preferred_element_type=jnp.float32)
           + lax.dot_general(y_ref[:], wr_ref[:], (((1,), (1,)), ((), ())),
                             preferred_element_type=jnp.float32)
           + b_ref[:])
    out_ref[:] = jnp.where(out >= 0.0, out, 0.01 * out)


def _layer2_body(agg_ref, cnt_ref, y_ref, wl_ref, wr_ref, b_ref, bat_ref,
                 node_ref, graph_ref):
    i = pl.program_id(0)
    rcp = 1.0 / jnp.maximum(cnt_ref[:], 1.0)
    nm = (lax.dot_general(agg_ref[:] * rcp, wl_ref[:],
                          (((1,), (1,)), ((), ())),
                          preferred_element_type=jnp.float32)
          + lax.dot_general(y_ref[:], wr_ref[:], (((1,), (1,)), ((), ())),
                            preferred_element_type=jnp.float32)
          + b_ref[:])
    node_ref[:] = nm
    onehot = (bat_ref[:] == lax.broadcasted_iota(jnp.int32, (BLK, G), 1)
              ).astype(jnp.float32)
    contrib = lax.dot_general(onehot, nm, (((0,), (0,)), ((), ())),
                              preferred_element_type=jnp.float32)

    @pl.when(i == 0)
    def _():
        graph_ref[:] = contrib

    @pl.when(i > 0)
    def _():
        graph_ref[:] += contrib


_COMMON_SPECS = [
    pl.BlockSpec((BLK, D), lambda i: (i, 0)),
    pl.BlockSpec((BLK, 1), lambda i: (i, 0)),
    pl.BlockSpec((BLK, D), lambda i: (i, 0)),
    pl.BlockSpec((D, D), lambda i: (0, 0)),
    pl.BlockSpec((D, D), lambda i: (0, 0)),
    pl.BlockSpec((1, D), lambda i: (0, 0)),
]

_layer1 = pl.pallas_call(
    _layer1_body,
    grid=(N // BLK,),
    in_specs=_COMMON_SPECS,
    out_specs=pl.BlockSpec((BLK, D), lambda i: (i, 0)),
    out_shape=jax.ShapeDtypeStruct((N, D), jnp.float32),
)

_layer2 = pl.pallas_call(
    _layer2_body,
    grid=(N // BLK,),
    in_specs=_COMMON_SPECS + [pl.BlockSpec((BLK, 1), lambda i: (i, 0))],
    out_specs=[
        pl.BlockSpec((BLK, D), lambda i: (i, 0)),
        pl.BlockSpec((G, D), lambda i: (0, 0)),
    ],
    out_shape=[
        jax.ShapeDtypeStruct((N, D), jnp.float32),
        jax.ShapeDtypeStruct((G, D), jnp.float32),
    ],
)


def kernel(x, edge_index, batch, W1_l, W1_r, b1, W2_l, W2_r, b2):
    src = edge_index[0]
    dst = edge_index[1]
    pad = EPAD - E
    src_pad = jnp.arange(pad, dtype=jnp.int32) % N
    dst_pad = DUMMY + jnp.arange(pad, dtype=jnp.int32) % (NPAD - N)
    src_p = jnp.concatenate([src, src_pad]).reshape(NS, NB, BATCH)
    src4 = jnp.stack([2 * src_p, 2 * src_p + 1])
    dst3 = jnp.concatenate([dst, dst_pad]).reshape(NS, NB, BATCH)
    z2 = jnp.zeros((CH, H), jnp.float32)
    z1 = jnp.zeros((CH,), jnp.float32)

    agg1, cnt = _sc_agg_cnt(x.reshape(2 * N, H), src4, dst3, z2, z1)
    cnt2 = cnt[:N].reshape(N, 1)
    h = _layer1(agg1, cnt2, x, W1_l, W1_r, b1.reshape(1, D))
    agg2 = _sc_agg_nocnt(h.reshape(2 * N, H), src4, dst3, z2)
    node_emb, graph_emb = _layer2(agg2, cnt2, h, W2_l, W2_r,
                                  b2.reshape(1, D), batch.reshape(N, 1))
    return node_emb, graph_emb


# speedup vs baseline: 1.0807x; 1.0006x over previous
import jax
import jax.numpy as jnp
from jax import lax
from jax.experimental import pallas as pl
from jax.experimental.pallas import tpu as pltpu
from jax.experimental.pallas import tpu_sc as plsc

N = 10000
E = 320000
D = 128
H = D // 2
G = 64

NC = 2
NS = 16
BATCH = 128
NBUF = 4
LOOK = 3
EPW = (E + NS - 1) // NS
NB = -(-EPW // (BATCH * NBUF)) * NBUF
EPAD = NS * NB * BATCH
CH = 640
NPAD = NS * CH
DUMMY = N

BLK = 2000


def _make_sc_agg(with_cnt):
    def body(*refs):
        if with_cnt:
            (x_hbm, src_hbm, dst_hbm, z2_hbm, z1_hbm, agg_out, cnt_out,
             src_v, dst_v, r0, r1, r2, r3, ones_v,
             agg_sh, cnt_sh, g0, g1, g2, g3) = refs
        else:
            (x_hbm, src_hbm, dst_hbm, z2_hbm, agg_out,
             src_v, dst_v, r0, r1, r2, r3, agg_sh,
             g0, g1, g2, g3) = refs
        rows = (r0, r1, r2, r3)
        gsem = (g0, g1, g2, g3)
        c = lax.axis_index("c")
        s = lax.axis_index("s")

        pltpu.sync_copy(z2_hbm, agg_sh.at[pl.ds(s * CH, CH)])
        pltpu.sync_copy(src_hbm.at[c, s], src_v)
        pltpu.sync_copy(dst_hbm.at[s], dst_v)

        if with_cnt:
            @pl.when(c == 0)
            def _():
                pltpu.sync_copy(z1_hbm, cnt_sh.at[pl.ds(s * CH, CH)])

            for i in range(BATCH // 16):
                ones_v[pl.ds(i * 16, 16)] = jnp.full((16,), 1.0, jnp.float32)

        for b in range(LOOK):
            pltpu.async_copy(x_hbm.at[src_v.at[b]], rows[b], gsem[b])
        plsc.subcore_barrier()

        @pl.loop(0, NB, step=NBUF)
        def _(gbase):
            for b in range(NBUF):
                i = gbase + b
                bn = (b + LOOK) % NBUF
                pltpu.make_async_copy(x_hbm.at[src_v.at[i]], rows[b],
                                      gsem[b]).wait()

                @pl.when(i + LOOK < NB)
                def _():
                    pltpu.async_copy(x_hbm.at[src_v.at[i + LOOK]], rows[bn],
                                     gsem[bn])

                pltpu.sync_copy(rows[b], agg_sh.at[dst_v.at[i]], add=True)

                if with_cnt:
                    @pl.when(c == 0)
                    def _():
                        pltpu.sync_copy(ones_v, cnt_sh.at[dst_v.at[i]],
                                        add=True)

        plsc.subcore_barrier()
        pltpu.sync_copy(agg_sh.at[pl.ds(s * CH, CH)],
                        agg_out.at[pl.ds(s * CH, CH), pl.ds(c * H, H)])

        if with_cnt:
            @pl.when(c == 0)
            def _():
                pltpu.sync_copy(cnt_sh.at[pl.ds(s * CH, CH)],
                                cnt_out.at[pl.ds(s * CH, CH)])

    out_type = jax.ShapeDtypeStruct((NPAD, D), jnp.float32)
    if with_cnt:
        out_type = [out_type, jax.ShapeDtypeStruct((NPAD,), jnp.float32)]
    scratch = [
        pltpu.VMEM((NB, BATCH), jnp.int32),
        pltpu.VMEM((NB, BATCH), jnp.int32),
    ] + [pltpu.VMEM((BATCH, H), jnp.float32)] * NBUF
    if with_cnt:
        scratch.append(pltpu.VMEM((BATCH,), jnp.float32))
    scratch.append(pltpu.VMEM_SHARED((NPAD, H), jnp.float32))
    if with_cnt:
        scratch.append(pltpu.VMEM_SHARED((NPAD,), jnp.float32))
    scratch += [pltpu.SemaphoreType.DMA] * NBUF
    return pl.kernel(
        body,
        out_type=out_type,
        mesh=plsc.VectorSubcoreMesh(core_axis_name="c", subcore_axis_name="s"),
        compiler_params=pltpu.CompilerParams(use_tc_tiling_on_sc=False,
                                             disable_bounds_checks=True),
        scratch_types=scratch,
    )


_sc_agg_cnt = _make_sc_agg(True)
_sc_agg_nocnt = _make_sc_agg(False)


def _layer1_body(agg_ref, cnt_ref, y_ref, wl_ref, wr_ref, b_ref, out_ref):
    rcp = 1.0 / jnp.maximum(cnt_ref[:], 1.0)
    out = (lax.dot_general(agg_ref[:] * rcp, wl_ref[:],
                           (((1,), (1,)), ((), ())),
                           preferred_element_type=jnp.float32)
           + lax.dot_general(y_ref[:], wr_ref[:], (((1,), (1,)), ((), ())),
                             preferred_element_type=jnp.float32)
           + b_ref[:])
    out_ref[:] = jnp.where(out >= 0.0, out, 0.01 * out)


def _layer2_body(agg_ref, cnt_ref, y_ref, wl_ref, wr_ref, b_ref, bat_ref,
                 node_ref, graph_ref):
    i = pl.program_id(0)
    rcp = 1.0 / jnp.maximum(cnt_ref[:], 1.0)
    nm = (lax.dot_general(agg_ref[:] * rcp, wl_ref[:],
                          (((1,), (1,)), ((), ())),
                          preferred_element_type=jnp.float32)
          + lax.dot_general(y_ref[:], wr_ref[:], (((1,), (1,)), ((), ())),
                            preferred_element_type=jnp.float32)
          + b_ref[:])
    node_ref[:] = nm
    onehot = (bat_ref[:] == lax.broadcasted_iota(jnp.int32, (BLK, G), 1)
              ).astype(jnp.float32)
    contrib = lax.dot_general(onehot, nm, (((0,), (0,)), ((), ())),
                              preferred_element_type=jnp.float32)

    @pl.when(i == 0)
    def _():
        graph_ref[:] = contrib

    @pl.when(i > 0)
    def _():
        graph_ref[:] += contrib


_COMMON_SPECS = [
    pl.BlockSpec((BLK, D), lambda i: (i, 0)),
    pl.BlockSpec((BLK, 1), lambda i: (i, 0)),
    pl.BlockSpec((BLK, D), lambda i: (i, 0)),
    pl.BlockSpec((D, D), lambda i: (0, 0)),
    pl.BlockSpec((D, D), lambda i: (0, 0)),
    pl.BlockSpec((1, D), lambda i: (0, 0)),
]

_layer1 = pl.pallas_call(
    _layer1_body,
    grid=(N // BLK,),
    in_specs=_COMMON_SPECS,
    out_specs=pl.BlockSpec((BLK, D), lambda i: (i, 0)),
    out_shape=jax.ShapeDtypeStruct((N, D), jnp.float32),
)

_layer2 = pl.pallas_call(
    _layer2_body,
    grid=(N // BLK,),
    in_specs=_COMMON_SPECS + [pl.BlockSpec((BLK, 1), lambda i: (i, 0))],
    out_specs=[
        pl.BlockSpec((BLK, D), lambda i: (i, 0)),
        pl.BlockSpec((G, D), lambda i: (0, 0)),
    ],
    out_shape=[
        jax.ShapeDtypeStruct((N, D), jnp.float32),
        jax.ShapeDtypeStruct((G, D), jnp.float32),
    ],
)


def kernel(x, edge_index, batch, W1_l, W1_r, b1, W2_l, W2_r, b2):
    src = edge_index[0]
    dst = edge_index[1]
    pad = EPAD - E
    src_pad = jnp.arange(pad, dtype=jnp.int32) % N
    dst_pad = DUMMY + jnp.arange(pad, dtype=jnp.int32) % (NPAD - N)
    src_p = jnp.concatenate([src, src_pad]).reshape(NS, NB, BATCH)
    src4 = jnp.stack([2 * src_p, 2 * src_p + 1])
    dst3 = jnp.concatenate([dst, dst_pad]).reshape(NS, NB, BATCH)
    z2 = jnp.zeros((CH, H), jnp.float32)
    z1 = jnp.zeros((CH,), jnp.float32)

    agg1, cnt = _sc_agg_cnt(x.reshape(2 * N, H), src4, dst3, z2, z1)
    cnt2 = cnt[:N].reshape(N, 1)
    h = _layer1(agg1, cnt2, x, W1_l, W1_r, b1.reshape(1, D))
    agg2 = _sc_agg_nocnt(h.reshape(2 * N, H), src4, dst3, z2)
    node_emb, graph_emb = _layer2(agg2, cnt2, h, W2_l, W2_r,
                                  b2.reshape(1, D), batch.reshape(N, 1))
    return node_emb, graph_emb
